# baseline (speedup 1.0000x reference)
import functools

import jax
import jax.numpy as jnp
from jax import lax
from jax.experimental import pallas as pl
from jax.experimental.pallas import tpu as pltpu
from jax.experimental.pallas import tpu_sc as plsc

B = 32768
D = 768
H = 256
OUT = 18
OUTP = 128
E = 8
NH = 4
BH = B // NH
GR = BH // 128
TILE = 512
NT = 24
NP = NT * TILE

NW = 32
RPW = BH // NW
XCH = 64
NXCH = RPW // XCH
YCH = 128
NYCH = RPW // YCH


def _route_body(g_ref, t_ref, p_ref, bexp_ref, srct_ref):
    g = g_ref[...]
    e = jnp.zeros(g.shape, jnp.int32)
    for k in range(E - 1):
        e += (g > t_ref[k]).astype(jnp.int32)

    ci = lax.broadcasted_iota(jnp.int32, (128, 128), 0)
    cj = lax.broadcasted_iota(jnp.int32, (128, 128), 1)
    tinc = (ci <= cj).astype(jnp.float32)
    ri = lax.broadcasted_iota(jnp.int32, (GR, GR), 0)
    rj = lax.broadcasted_iota(jnp.int32, (GR, GR), 1)
    slow = (rj < ri).astype(jnp.float32)

    tv = lax.broadcasted_iota(jnp.int32, (1, 256), 1)
    p_halves = []
    bexp_halves = []
    srct_halves = []
    for hh in range(NH):
        eh = e[hh * GR:(hh + 1) * GR]
        p_acc = jnp.zeros((GR, 128), jnp.float32)
        pstart = jnp.int32(0)
        acc = jnp.full((1, 256), -1, jnp.int32)
        for ex in range(E):
            h = (eh == ex).astype(jnp.float32)
            rowcum = jnp.dot(h, tinc, preferred_element_type=jnp.float32)
            excl = rowcum - h
            rowsum = rowcum[:, 127:128]
            rowpre = jnp.dot(slow, jnp.broadcast_to(rowsum, (GR, 128)),
                             preferred_element_type=jnp.float32)
            cnt = jnp.sum(h).astype(jnp.int32)
            rank = excl + rowpre
            p_acc += h * (rank + pstart.astype(jnp.float32))
            acc += (tv >= pstart // TILE).astype(jnp.int32)
            pstart = pstart + ((cnt + TILE - 1) // TILE) * TILE
        p_halves.append(p_acc.astype(jnp.int32))
        bexp_halves.append(acc)
        srct_halves.append(jnp.minimum(tv, pstart // TILE - 1))
    p_ref[...] = jnp.concatenate(p_halves, axis=0)
    bexp_ref[...] = jnp.concatenate(bexp_halves, axis=0)
    srct_ref[...] = jnp.concatenate(srct_halves, axis=0)


def _route(noise2d, thres):
    return pl.pallas_call(
        _route_body,
        in_specs=[
            pl.BlockSpec(memory_space=pltpu.VMEM),
            pl.BlockSpec(memory_space=pltpu.SMEM),
        ],
        out_specs=[
            pl.BlockSpec(memory_space=pltpu.VMEM),
            pl.BlockSpec(memory_space=pltpu.VMEM),
            pl.BlockSpec(memory_space=pltpu.VMEM),
        ],
        out_shape=[
            jax.ShapeDtypeStruct((256, 128), jnp.int32),
            jax.ShapeDtypeStruct((NH, 256), jnp.int32),
            jax.ShapeDtypeStruct((NH, 256), jnp.int32),
        ],
    )(noise2d, thres)


@functools.lru_cache(maxsize=None)
def _sc_kernels(hh):
    mesh = plsc.VectorSubcoreMesh(core_axis_name="c", subcore_axis_name="s")

    @functools.partial(
        pl.kernel,
        out_type=jax.ShapeDtypeStruct((NP, D), jnp.float32),
        mesh=mesh,
        scratch_types=[
            pltpu.VMEM((NXCH, XCH), jnp.int32),
            pltpu.VMEM((XCH, D), jnp.float32),
            pltpu.VMEM((XCH, D), jnp.float32),
            pltpu.SemaphoreType.DMA,
            pltpu.SemaphoreType.DMA,
            pltpu.SemaphoreType.DMA,
            pltpu.SemaphoreType.DMA,
        ],
    )
    def scatter_x(x_hbm, p_hbm, xs_hbm, idx_v, xb0, xb1, sl0, sl1, ss0, ss1):
        wid = lax.axis_index("s") * 2 + lax.axis_index("c")
        base = hh * BH + wid * RPW
        pltpu.sync_copy(p_hbm.at[wid], idx_v)
        xbs = (xb0, xb1)
        sls = (sl0, sl1)
        sss = (ss0, ss1)
        loads = [None] * NXCH
        scats = [None] * NXCH
        loads[0] = pltpu.async_copy(x_hbm.at[pl.ds(base, XCH)], xb0, sl0)
        for ch in range(NXCH):
            loads[ch].wait()
            scats[ch] = pltpu.async_copy(xbs[ch % 2],
                                         xs_hbm.at[idx_v.at[ch]],
                                         sss[ch % 2])
            if ch + 1 < NXCH:
                if ch >= 1:
                    scats[ch - 1].wait()
                loads[ch + 1] = pltpu.async_copy(
                    x_hbm.at[pl.ds(base + (ch + 1) * XCH, XCH)],
                    xbs[(ch + 1) % 2], sls[(ch + 1) % 2])
        scats[NXCH - 2].wait()
        scats[NXCH - 1].wait()

    @functools.partial(
        pl.kernel,
        out_type=jax.ShapeDtypeStruct((BH, OUTP), jnp.float32),
        mesh=mesh,
        scratch_types=[
            pltpu.VMEM((NYCH, YCH), jnp.int32),
            pltpu.VMEM((YCH, OUTP), jnp.float32),
            pltpu.SemaphoreType.DMA,
        ],
    )
    def gather_y(ys_hbm, p_hbm, out_hbm, idx_v, yb, sem):
        wid = lax.axis_index("s") * 2 + lax.axis_index("c")
        base = wid * RPW
        pltpu.sync_copy(p_hbm.at[wid], idx_v)
        for j in range(NYCH):
            pltpu.async_copy(ys_hbm.at[idx_v.at[j]], yb, sem).wait()
            pltpu.sync_copy(yb, out_hbm.at[pl.ds(base + j * YCH, YCH)])

    return scatter_x, gather_y


def _mlp_body(bexp_ref, srct_ref, xs_ref, w1_ref, b1_ref, w2_ref, b2_ref,
              w3_ref, b3_ref, ys_ref):
    t = pl.program_id(0)

    @pl.when(srct_ref[t] == t)
    def _():
        bf = jnp.bfloat16
        x = xs_ref[...].astype(bf)
        h = jnp.tanh(jnp.dot(x, w1_ref[0],
                             preferred_element_type=jnp.float32) + b1_ref[0])
        h = jnp.tanh(jnp.dot(h.astype(bf), w2_ref[0],
                             preferred_element_type=jnp.float32) + b2_ref[0])
        ys_ref[...] = (jnp.dot(h.astype(bf), w3_ref[0],
                               preferred_element_type=jnp.float32) + b3_ref[0])


def _mlp(bexp, srct, xs, w1, b1, w2, b2, w3p, b3p):
    grid_spec = pltpu.PrefetchScalarGridSpec(
        num_scalar_prefetch=2,
        grid=(NT,),
        in_specs=[
            pl.BlockSpec((TILE, D), lambda t, be, st: (st[t], 0)),
            pl.BlockSpec((1, D, H), lambda t, be, st: (be[st[t]], 0, 0)),
            pl.BlockSpec((1, 1, H), lambda t, be, st: (be[st[t]], 0, 0)),
            pl.BlockSpec((1, H, H), lambda t, be, st: (be[st[t]], 0, 0)),
            pl.BlockSpec((1, 1, H), lambda t, be, st: (be[st[t]], 0, 0)),
            pl.BlockSpec((1, H, OUTP), lambda t, be, st: (be[st[t]], 0, 0)),
            pl.BlockSpec((1, 1, OUTP), lambda t, be, st: (be[st[t]], 0, 0)),
        ],
        out_specs=pl.BlockSpec((TILE, OUTP), lambda t, be, st: (st[t], 0)),
    )
    return pl.pallas_call(
        _mlp_body,
        grid_spec=grid_spec,
        out_shape=jax.ShapeDtypeStruct((NP, OUTP), jnp.float32),
    )(bexp, srct, xs, w1, b1, w2, b2, w3p, b3p)


def kernel(original_obs, random_noise, W1, b1, W2, b2, W3, b3):
    ps = jnp.arange(1, E, dtype=jnp.float32) / E
    thres = jnp.sqrt(2.0) * jax.scipy.special.erfinv(2.0 * ps - 1.0)

    g2d = random_noise.reshape(256, 128)
    p2d, bexp2d, srct2d = _route(g2d, thres)
    p = p2d.reshape(B)

    bf = jnp.bfloat16
    w1b = W1.astype(bf)
    w2b = W2.astype(bf)
    w3p = jnp.pad(W3, ((0, 0), (0, 0), (0, OUTP - OUT))).astype(bf)
    b1r = b1.reshape(E, 1, H)
    b2r = b2.reshape(E, 1, H)
    b3r = jnp.pad(b3, ((0, 0), (0, OUTP - OUT))).reshape(E, 1, OUTP)

    outs = []
    for hh in range(NH):
        scatter_x, gather_y = _sc_kernels(hh)
        p_h = p[hh * BH:(hh + 1) * BH]
        bexp_h = bexp2d[hh, :NT]
        srct_h = srct2d[hh, :NT]
        xs = scatter_x(original_obs, p_h.reshape(NW, NXCH, XCH))
        ys = _mlp(bexp_h, srct_h, xs, w1b, b1r, w2b, b2r, w3p, b3r)
        outs.append(gather_y(ys, p_h.reshape(NW, NYCH, YCH))[:, :OUT])
    return jnp.concatenate(outs, axis=0)


# speedup vs baseline: 1.0966x; 1.0966x over previous
import functools

import jax
import jax.numpy as jnp
from jax import lax
from jax.experimental import pallas as pl
from jax.experimental.pallas import tpu as pltpu
from jax.experimental.pallas import tpu_sc as plsc

B = 32768
D = 768
H = 256
OUT = 18
OUTP = 128
E = 8
NH = 2
BH = B // NH
GR = BH // 128
TILE = 1024
NT = 24
NP = NT * TILE

NW = 32
RPW = BH // NW
XCH = 64
NXCH = RPW // XCH
YCH = 128
NYCH = RPW // YCH


def _route_body(g_ref, t_ref, p_ref, bexp_ref, srct_ref):
    g = g_ref[...]
    e = jnp.zeros(g.shape, jnp.int32)
    for k in range(E - 1):
        e += (g > t_ref[k]).astype(jnp.int32)

    ci = lax.broadcasted_iota(jnp.int32, (128, 128), 0)
    cj = lax.broadcasted_iota(jnp.int32, (128, 128), 1)
    tinc = (ci <= cj).astype(jnp.float32)
    ri = lax.broadcasted_iota(jnp.int32, (GR, GR), 0)
    rj = lax.broadcasted_iota(jnp.int32, (GR, GR), 1)
    slow = (rj < ri).astype(jnp.float32)

    tv = lax.broadcasted_iota(jnp.int32, (1, 256), 1)
    p_halves = []
    bexp_halves = []
    srct_halves = []
    for hh in range(NH):
        eh = e[hh * GR:(hh + 1) * GR]
        p_acc = jnp.zeros((GR, 128), jnp.float32)
        pstart = jnp.int32(0)
        acc = jnp.full((1, 256), -1, jnp.int32)
        for ex in range(E):
            h = (eh == ex).astype(jnp.float32)
            rowcum = jnp.dot(h, tinc, preferred_element_type=jnp.float32)
            excl = rowcum - h
            rowsum = rowcum[:, 127:128]
            rowpre = jnp.dot(slow, jnp.broadcast_to(rowsum, (GR, 128)),
                             preferred_element_type=jnp.float32)
            cnt = jnp.sum(h).astype(jnp.int32)
            rank = excl + rowpre
            p_acc += h * (rank + pstart.astype(jnp.float32))
            acc += (tv >= pstart // TILE).astype(jnp.int32)
            pstart = pstart + ((cnt + TILE - 1) // TILE) * TILE
        p_halves.append(p_acc.astype(jnp.int32))
        bexp_halves.append(acc)
        srct_halves.append(jnp.minimum(tv, pstart // TILE - 1))
    p_ref[...] = jnp.concatenate(p_halves, axis=0)
    bexp_ref[...] = jnp.concatenate(bexp_halves, axis=0)
    srct_ref[...] = jnp.concatenate(srct_halves, axis=0)


def _route(noise2d, thres):
    return pl.pallas_call(
        _route_body,
        in_specs=[
            pl.BlockSpec(memory_space=pltpu.VMEM),
            pl.BlockSpec(memory_space=pltpu.SMEM),
        ],
        out_specs=[
            pl.BlockSpec(memory_space=pltpu.VMEM),
            pl.BlockSpec(memory_space=pltpu.VMEM),
            pl.BlockSpec(memory_space=pltpu.VMEM),
        ],
        out_shape=[
            jax.ShapeDtypeStruct((256, 128), jnp.int32),
            jax.ShapeDtypeStruct((NH, 256), jnp.int32),
            jax.ShapeDtypeStruct((NH, 256), jnp.int32),
        ],
    )(noise2d, thres)


@functools.lru_cache(maxsize=None)
def _sc_kernels(hh):
    mesh = plsc.VectorSubcoreMesh(core_axis_name="c", subcore_axis_name="s")

    @functools.partial(
        pl.kernel,
        out_type=jax.ShapeDtypeStruct((NP, D), jnp.float32),
        mesh=mesh,
        scratch_types=[
            pltpu.VMEM((NXCH, XCH), jnp.int32),
            pltpu.VMEM((XCH, D), jnp.float32),
            pltpu.VMEM((XCH, D), jnp.float32),
            pltpu.SemaphoreType.DMA,
            pltpu.SemaphoreType.DMA,
            pltpu.SemaphoreType.DMA,
            pltpu.SemaphoreType.DMA,
        ],
    )
    def scatter_x(x_hbm, p_hbm, xs_hbm, idx_v, xb0, xb1, sl0, sl1, ss0, ss1):
        wid = lax.axis_index("s") * 2 + lax.axis_index("c")
        base = hh * BH + wid * RPW
        pltpu.sync_copy(p_hbm.at[wid], idx_v)
        xbs = (xb0, xb1)
        sls = (sl0, sl1)
        sss = (ss0, ss1)
        loads = [None] * NXCH
        scats = [None] * NXCH
        loads[0] = pltpu.async_copy(x_hbm.at[pl.ds(base, XCH)], xb0, sl0)
        for ch in range(NXCH):
            loads[ch].wait()
            scats[ch] = pltpu.async_copy(xbs[ch % 2],
                                         xs_hbm.at[idx_v.at[ch]],
                                         sss[ch % 2])
            if ch + 1 < NXCH:
                if ch >= 1:
                    scats[ch - 1].wait()
                loads[ch + 1] = pltpu.async_copy(
                    x_hbm.at[pl.ds(base + (ch + 1) * XCH, XCH)],
                    xbs[(ch + 1) % 2], sls[(ch + 1) % 2])
        scats[NXCH - 2].wait()
        scats[NXCH - 1].wait()

    @functools.partial(
        pl.kernel,
        out_type=jax.ShapeDtypeStruct((BH, OUTP), jnp.float32),
        mesh=mesh,
        scratch_types=[
            pltpu.VMEM((NYCH, YCH), jnp.int32),
            pltpu.VMEM((YCH, OUTP), jnp.float32),
            pltpu.SemaphoreType.DMA,
        ],
    )
    def gather_y(ys_hbm, p_hbm, out_hbm, idx_v, yb, sem):
        wid = lax.axis_index("s") * 2 + lax.axis_index("c")
        base = wid * RPW
        pltpu.sync_copy(p_hbm.at[wid], idx_v)
        for j in range(NYCH):
            pltpu.async_copy(ys_hbm.at[idx_v.at[j]], yb, sem).wait()
            pltpu.sync_copy(yb, out_hbm.at[pl.ds(base + j * YCH, YCH)])

    return scatter_x, gather_y


def _mlp_body(bexp_ref, srct_ref, xa_ref, xb_ref, w1_ref, b1_ref, w2_ref,
              b2_ref, w3_ref, b3_ref, ys_ref):
    t = pl.program_id(0)

    @pl.when(srct_ref[t] == t)
    def _():
        bf = jnp.bfloat16
        xa = xa_ref[...].astype(bf)
        xb = xb_ref[...].astype(bf)
        h = jnp.tanh(jnp.dot(xa, w1_ref[0, 0],
                             preferred_element_type=jnp.float32)
                     + jnp.dot(xb, w1_ref[0, 1],
                               preferred_element_type=jnp.float32)
                     + b1_ref[0])
        h = jnp.tanh(jnp.dot(h.astype(bf), w2_ref[0],
                             preferred_element_type=jnp.float32) + b2_ref[0])
        ys_ref[...] = (jnp.dot(h.astype(bf), w3_ref[0],
                               preferred_element_type=jnp.float32) + b3_ref[0])


def _mlp(bexp, srct, xs, w1, b1, w2, b2, w3p, b3p):
    grid_spec = pltpu.PrefetchScalarGridSpec(
        num_scalar_prefetch=2,
        grid=(NT,),
        in_specs=[
            pl.BlockSpec((TILE, D // 2), lambda t, be, st: (st[t], 0)),
            pl.BlockSpec((TILE, D // 2), lambda t, be, st: (st[t], 1)),
            pl.BlockSpec((1, 2, D // 2, H),
                         lambda t, be, st: (be[st[t]], 0, 0, 0)),
            pl.BlockSpec((1, 1, H), lambda t, be, st: (be[st[t]], 0, 0)),
            pl.BlockSpec((1, H, H), lambda t, be, st: (be[st[t]], 0, 0)),
            pl.BlockSpec((1, 1, H), lambda t, be, st: (be[st[t]], 0, 0)),
            pl.BlockSpec((1, H, OUTP), lambda t, be, st: (be[st[t]], 0, 0)),
            pl.BlockSpec((1, 1, OUTP), lambda t, be, st: (be[st[t]], 0, 0)),
        ],
        out_specs=pl.BlockSpec((TILE, OUTP), lambda t, be, st: (st[t], 0)),
    )
    return pl.pallas_call(
        _mlp_body,
        grid_spec=grid_spec,
        out_shape=jax.ShapeDtypeStruct((NP, OUTP), jnp.float32),
    )(bexp, srct, xs, xs, w1, b1, w2, b2, w3p, b3p)


def kernel(original_obs, random_noise, W1, b1, W2, b2, W3, b3):
    ps = jnp.arange(1, E, dtype=jnp.float32) / E
    thres = jnp.sqrt(2.0) * jax.scipy.special.erfinv(2.0 * ps - 1.0)

    g2d = random_noise.reshape(256, 128)
    p2d, bexp2d, srct2d = _route(g2d, thres)
    p = p2d.reshape(B)

    bf = jnp.bfloat16
    w1b = W1.reshape(E, 2, D // 2, H).astype(bf)
    w2b = W2.astype(bf)
    w3p = jnp.pad(W3, ((0, 0), (0, 0), (0, OUTP - OUT))).astype(bf)
    b1r = b1.reshape(E, 1, H)
    b2r = b2.reshape(E, 1, H)
    b3r = jnp.pad(b3, ((0, 0), (0, OUTP - OUT))).reshape(E, 1, OUTP)

    outs = []
    for hh in range(NH):
        scatter_x, gather_y = _sc_kernels(hh)
        p_h = p[hh * BH:(hh + 1) * BH]
        bexp_h = bexp2d[hh, :NT]
        srct_h = srct2d[hh, :NT]
        xs = scatter_x(original_obs, p_h.reshape(NW, NXCH, XCH))
        ys = _mlp(bexp_h, srct_h, xs, w1b, b1r, w2b, b2r, w3p, b3r)
        outs.append(gather_y(ys, p_h.reshape(NW, NYCH, YCH))[:, :OUT])
    return jnp.concatenate(outs, axis=0)


# speedup vs baseline: 1.1448x; 1.0439x over previous
import functools

import jax
import jax.numpy as jnp
from jax import lax
from jax.experimental import pallas as pl
from jax.experimental.pallas import tpu as pltpu
from jax.experimental.pallas import tpu_sc as plsc

B = 32768
D = 768
H = 256
OUT = 18
OUTP = 128
E = 8
NH = 2
BH = B // NH
GR = BH // 128
TILE = 1024
NT = 24
NP = NT * TILE

NW = 32
RPW = BH // NW
XCH = 32
NXCH = RPW // XCH
NBUF = 4
YCH = 128
NYCH = RPW // YCH


def _route_body(g_ref, t_ref, p_ref, bexp_ref, srct_ref):
    g = g_ref[...]
    e = jnp.zeros(g.shape, jnp.int32)
    for k in range(E - 1):
        e += (g > t_ref[k]).astype(jnp.int32)

    ci = lax.broadcasted_iota(jnp.int32, (128, 128), 0)
    cj = lax.broadcasted_iota(jnp.int32, (128, 128), 1)
    tinc = (ci <= cj).astype(jnp.float32)
    ri = lax.broadcasted_iota(jnp.int32, (GR, GR), 0)
    rj = lax.broadcasted_iota(jnp.int32, (GR, GR), 1)
    slow = (rj < ri).astype(jnp.float32)

    tv = lax.broadcasted_iota(jnp.int32, (1, 256), 1)
    p_halves = []
    bexp_halves = []
    srct_halves = []
    for hh in range(NH):
        eh = e[hh * GR:(hh + 1) * GR]
        p_acc = jnp.zeros((GR, 128), jnp.float32)
        pstart = jnp.int32(0)
        acc = jnp.full((1, 256), -1, jnp.int32)
        for ex in range(E):
            h = (eh == ex).astype(jnp.float32)
            rowcum = jnp.dot(h, tinc, preferred_element_type=jnp.float32)
            excl = rowcum - h
            rowsum = rowcum[:, 127:128]
            rowpre = jnp.dot(slow, jnp.broadcast_to(rowsum, (GR, 128)),
                             preferred_element_type=jnp.float32)
            cnt = jnp.sum(h).astype(jnp.int32)
            rank = excl + rowpre
            p_acc += h * (rank + pstart.astype(jnp.float32))
            acc += (tv >= pstart // TILE).astype(jnp.int32)
            pstart = pstart + ((cnt + TILE - 1) // TILE) * TILE
        p_halves.append(p_acc.astype(jnp.int32))
        bexp_halves.append(acc)
        srct_halves.append(jnp.minimum(tv, pstart // TILE - 1))
    p_ref[...] = jnp.concatenate(p_halves, axis=0)
    bexp_ref[...] = jnp.concatenate(bexp_halves, axis=0)
    srct_ref[...] = jnp.concatenate(srct_halves, axis=0)


def _route(noise2d, thres):
    return pl.pallas_call(
        _route_body,
        in_specs=[
            pl.BlockSpec(memory_space=pltpu.VMEM),
            pl.BlockSpec(memory_space=pltpu.SMEM),
        ],
        out_specs=[
            pl.BlockSpec(memory_space=pltpu.VMEM),
            pl.BlockSpec(memory_space=pltpu.VMEM),
            pl.BlockSpec(memory_space=pltpu.VMEM),
        ],
        out_shape=[
            jax.ShapeDtypeStruct((256, 128), jnp.int32),
            jax.ShapeDtypeStruct((NH, 256), jnp.int32),
            jax.ShapeDtypeStruct((NH, 256), jnp.int32),
        ],
    )(noise2d, thres)


@functools.lru_cache(maxsize=None)
def _sc_kernels(hh):
    mesh = plsc.VectorSubcoreMesh(core_axis_name="c", subcore_axis_name="s")

    @functools.partial(
        pl.kernel,
        out_type=jax.ShapeDtypeStruct((NP, D), jnp.float32),
        mesh=mesh,
        scratch_types=(
            [pltpu.VMEM((NXCH, XCH), jnp.int32)]
            + [pltpu.VMEM((XCH, D), jnp.float32) for _ in range(NBUF)]
            + [pltpu.SemaphoreType.DMA for _ in range(2 * NBUF)]
        ),
    )
    def scatter_x(x_hbm, p_hbm, xs_hbm, idx_v, *bufs):
        xbs = bufs[:NBUF]
        sls = bufs[NBUF:2 * NBUF]
        sss = bufs[2 * NBUF:3 * NBUF]
        wid = lax.axis_index("s") * 2 + lax.axis_index("c")
        base = hh * BH + wid * RPW
        pltpu.sync_copy(p_hbm.at[wid], idx_v)
        loads = [None] * NXCH
        scats = [None] * NXCH
        for ch in range(NBUF - 1):
            loads[ch] = pltpu.async_copy(
                x_hbm.at[pl.ds(base + ch * XCH, XCH)], xbs[ch % NBUF],
                sls[ch % NBUF])
        for ch in range(NXCH):
            loads[ch].wait()
            scats[ch] = pltpu.async_copy(xbs[ch % NBUF],
                                         xs_hbm.at[idx_v.at[ch]],
                                         sss[ch % NBUF])
            nx = ch + NBUF - 1
            if nx < NXCH:
                if nx >= NBUF:
                    scats[nx - NBUF].wait()
                loads[nx] = pltpu.async_copy(
                    x_hbm.at[pl.ds(base + nx * XCH, XCH)],
                    xbs[nx % NBUF], sls[nx % NBUF])
        for ch in range(max(0, NXCH - NBUF), NXCH):
            scats[ch].wait()

    @functools.partial(
        pl.kernel,
        out_type=jax.ShapeDtypeStruct((BH, OUTP), jnp.float32),
        mesh=mesh,
        scratch_types=(
            [pltpu.VMEM((NYCH, YCH), jnp.int32)]
            + [pltpu.VMEM((YCH, OUTP), jnp.float32) for _ in range(NYCH)]
            + [pltpu.SemaphoreType.DMA for _ in range(NYCH)]
        ),
    )
    def gather_y(ys_hbm, p_hbm, out_hbm, idx_v, *bufs):
        ybs = bufs[:NYCH]
        sems = bufs[NYCH:2 * NYCH]
        wid = lax.axis_index("s") * 2 + lax.axis_index("c")
        base = wid * RPW
        pltpu.sync_copy(p_hbm.at[wid], idx_v)
        gs = [pltpu.async_copy(ys_hbm.at[idx_v.at[j]], ybs[j], sems[j])
              for j in range(NYCH)]
        for j in range(NYCH):
            gs[j].wait()
            pltpu.sync_copy(ybs[j], out_hbm.at[pl.ds(base + j * YCH, YCH)])

    return scatter_x, gather_y


def _mlp_body(bexp_ref, srct_ref, xs_ref, w1_ref, b1_ref, w2_ref, b2_ref,
              w3_ref, b3_ref, ys_ref):
    t = pl.program_id(0)

    @pl.when(srct_ref[t] == t)
    def _():
        bf = jnp.bfloat16
        x = xs_ref[...].astype(bf)
        h = jnp.tanh(jnp.dot(x, w1_ref[0],
                             preferred_element_type=jnp.float32) + b1_ref[0])
        h = jnp.tanh(jnp.dot(h.astype(bf), w2_ref[0],
                             preferred_element_type=jnp.float32) + b2_ref[0])
        ys_ref[...] = (jnp.dot(h.astype(bf), w3_ref[0],
                               preferred_element_type=jnp.float32) + b3_ref[0])


def _mlp(bexp, srct, xs, w1, b1, w2, b2, w3p, b3p):
    grid_spec = pltpu.PrefetchScalarGridSpec(
        num_scalar_prefetch=2,
        grid=(NT,),
        in_specs=[
            pl.BlockSpec((TILE, D), lambda t, be, st: (st[t], 0)),
            pl.BlockSpec((1, D, H), lambda t, be, st: (be[st[t]], 0, 0)),
            pl.BlockSpec((1, 1, H), lambda t, be, st: (be[st[t]], 0, 0)),
            pl.BlockSpec((1, H, H), lambda t, be, st: (be[st[t]], 0, 0)),
            pl.BlockSpec((1, 1, H), lambda t, be, st: (be[st[t]], 0, 0)),
            pl.BlockSpec((1, H, OUTP), lambda t, be, st: (be[st[t]], 0, 0)),
            pl.BlockSpec((1, 1, OUTP), lambda t, be, st: (be[st[t]], 0, 0)),
        ],
        out_specs=pl.BlockSpec((TILE, OUTP), lambda t, be, st: (st[t], 0)),
    )
    return pl.pallas_call(
        _mlp_body,
        grid_spec=grid_spec,
        out_shape=jax.ShapeDtypeStruct((NP, OUTP), jnp.float32),
    )(bexp, srct, xs, w1, b1, w2, b2, w3p, b3p)


def kernel(original_obs, random_noise, W1, b1, W2, b2, W3, b3):
    ps = jnp.arange(1, E, dtype=jnp.float32) / E
    thres = jnp.sqrt(2.0) * jax.scipy.special.erfinv(2.0 * ps - 1.0)

    g2d = random_noise.reshape(256, 128)
    p2d, bexp2d, srct2d = _route(g2d, thres)
    p = p2d.reshape(B)

    bf = jnp.bfloat16
    w1b = W1.astype(bf)
    w2b = W2.astype(bf)
    w3p = jnp.pad(W3, ((0, 0), (0, 0), (0, OUTP - OUT))).astype(bf)
    b1r = b1.reshape(E, 1, H)
    b2r = b2.reshape(E, 1, H)
    b3r = jnp.pad(b3, ((0, 0), (0, OUTP - OUT))).reshape(E, 1, OUTP)

    outs = []
    for hh in range(NH):
        scatter_x, gather_y = _sc_kernels(hh)
        p_h = p[hh * BH:(hh + 1) * BH]
        bexp_h = bexp2d[hh, :NT]
        srct_h = srct2d[hh, :NT]
        xs = scatter_x(original_obs, p_h.reshape(NW, NXCH, XCH))
        ys = _mlp(bexp_h, srct_h, xs, w1b, b1r, w2b, b2r, w3p, b3r)
        outs.append(gather_y(ys, p_h.reshape(NW, NYCH, YCH))[:, :OUT])
    return jnp.concatenate(outs, axis=0)
